# Initial kernel scaffold; baseline (speedup 1.0000x reference)
import jax
import jax.numpy as jnp
from jax.experimental import pallas as pl


def kernel(x, table, mat):
    raise NotImplementedError("write your pallas kernel here")


# speedup vs baseline: 1.0475x; 1.0475x over previous
import functools

import jax
import jax.numpy as jnp
from jax import lax
from jax.experimental import pallas as pl
from jax.experimental.pallas import tpu as pltpu
from jax.experimental.pallas import tpu_sc as plsc

D = 64
CH = 128
NBUF = 8
NC = 2
NS = 16
NW = NC * NS


def _gather_rows(table, xf2):
    R = xf2.shape[0]
    rows_per_w = R // NW
    groups = rows_per_w // NBUF
    mesh = plsc.VectorSubcoreMesh(core_axis_name="c", subcore_axis_name="s")

    @functools.partial(
        pl.kernel,
        mesh=mesh,
        out_type=jax.ShapeDtypeStruct((R, CH, D), jnp.float32),
        scratch_types=[
            pltpu.VMEM((NBUF, CH), jnp.int32),
            pltpu.VMEM((NBUF, CH, D), jnp.float32),
            pltpu.SemaphoreType.DMA,
            pltpu.SemaphoreType.DMA,
        ],
        compiler_params=pltpu.CompilerParams(use_tc_tiling_on_sc=False),
    )
    def k(table_hbm, xf_hbm, e_hbm, idx_v, rows_v, gsem, osem):
        wid = lax.axis_index("s") * NC + lax.axis_index("c")
        row0 = wid * rows_per_w

        @pl.loop(0, groups)
        def group(g):
            base = row0 + g * NBUF
            pltpu.sync_copy(xf_hbm.at[pl.ds(base, NBUF)], idx_v)
            gathers = [
                pltpu.async_copy(table_hbm.at[idx_v.at[b]], rows_v.at[b], gsem)
                for b in range(NBUF)
            ]
            stores = []
            for b in range(NBUF):
                gathers[b].wait()
                stores.append(
                    pltpu.async_copy(rows_v.at[b], e_hbm.at[base + b], osem))
            for s in stores:
                s.wait()

    return k(table, xf2)


def _project(e2, mat):
    M = e2.shape[0]
    BM = 2048

    def mm(e_ref, m_ref, o_ref):
        o_ref[...] = lax.dot_general(
            e_ref[...], m_ref[...],
            (((1,), (1,)), ((), ())),
            preferred_element_type=jnp.float32)

    return pl.pallas_call(
        mm,
        grid=(M // BM,),
        in_specs=[
            pl.BlockSpec((BM, D), lambda i: (i, 0)),
            pl.BlockSpec((D, D), lambda i: (0, 0)),
        ],
        out_specs=pl.BlockSpec((BM, D), lambda i: (i, 0)),
        out_shape=jax.ShapeDtypeStruct((M, D), jnp.float32),
    )(e2, mat)


def kernel(x, table, mat):
    batch, length = x.shape
    xf2 = x.reshape(-1).astype(jnp.int32).reshape(-1, CH)
    e = _gather_rows(table, xf2)
    t = _project(e.reshape(-1, D), mat)
    return t.reshape(batch, length, -1)


# speedup vs baseline: 1.2625x; 1.2052x over previous
import functools

import jax
import jax.numpy as jnp
from jax import lax
from jax.experimental import pallas as pl
from jax.experimental.pallas import tpu as pltpu
from jax.experimental.pallas import tpu_sc as plsc

D = 64
NC = 2
NS = 16
NW = NC * NS
RB = 8
CP = 512


def _project_table(tableT, mat):
    V = tableT.shape[1]
    nb = pl.cdiv(V, 2 * CP)

    def body(t_ref, m_ref, o_ref):
        dn = (((0,), (1,)), ((), ()))
        e = lax.dot_general(t_ref[:, :CP], m_ref[...], dn,
                            preferred_element_type=jnp.float32)
        o = lax.dot_general(t_ref[:, CP:], m_ref[...], dn,
                            preferred_element_type=jnp.float32)
        o_ref[...] = jnp.concatenate([e, o], axis=1)

    return pl.pallas_call(
        body,
        grid=(nb,),
        in_specs=[
            pl.BlockSpec((D, 2 * CP), lambda i: (0, i)),
            pl.BlockSpec((D, D), lambda i: (0, 0)),
        ],
        out_specs=pl.BlockSpec((CP, 2 * D), lambda i: (i, 0)),
        out_shape=jax.ShapeDtypeStruct((nb * CP, 2 * D), jnp.float32),
    )(tableT, mat)


def _gather_rows(p, x):
    B, L = x.shape
    rows_per_w = B // NW
    mesh = plsc.VectorSubcoreMesh(core_axis_name="c", subcore_axis_name="s")

    @functools.partial(
        pl.kernel,
        mesh=mesh,
        out_type=jax.ShapeDtypeStruct((B, L, D), jnp.float32),
        scratch_types=[
            pltpu.VMEM((RB, L), jnp.int32),
            pltpu.VMEM((RB, L, D), jnp.float32),
            pltpu.SemaphoreType.DMA,
            pltpu.SemaphoreType.DMA,
        ],
        compiler_params=pltpu.CompilerParams(use_tc_tiling_on_sc=False),
    )
    def k(p_hbm, x_hbm, out_hbm, idx_v, rows_v, gsem, osem):
        wid = lax.axis_index("s") * NC + lax.axis_index("c")
        row0 = wid * rows_per_w

        @pl.loop(0, rows_per_w // RB)
        def group(g):
            base = row0 + g * RB
            pltpu.sync_copy(x_hbm.at[pl.ds(base, RB)], idx_v)
            gathers = []
            for b in range(RB):
                gathers.append(pltpu.async_copy(
                    p_hbm.at[idx_v.at[b, pl.ds(0, 128)]],
                    rows_v.at[b, pl.ds(0, 128)], gsem))
                gathers.append(pltpu.async_copy(
                    p_hbm.at[idx_v.at[b, pl.ds(128, L - 128)]],
                    rows_v.at[b, pl.ds(128, L - 128)], gsem))
            stores = []
            for b in range(RB):
                gathers[2 * b].wait()
                gathers[2 * b + 1].wait()
                stores.append(
                    pltpu.async_copy(rows_v.at[b], out_hbm.at[base + b], osem))
            for s in stores:
                s.wait()

    return k(p, x)


def kernel(x, table, mat):
    ppack = _project_table(table.T, mat)
    p = ppack.reshape(-1, D)
    xg = x.astype(jnp.int32)
    xg = (xg & ~jnp.int32(1023)) + ((xg & 511) << 1) + ((xg >> 9) & 1)
    return _gather_rows(p, xg)
